# baseline (speedup 1.0000x reference)
import functools

import jax
import jax.numpy as jnp
from jax import lax
from jax.experimental import pallas as pl
from jax.experimental.pallas import tpu as pltpu
from jax.experimental.pallas import tpu_sc as plsc

_B, _C, _H, _W = 16, 64, 32, 32
_HW = _H * _W
_K = 512
_MB = 4

_NC = 2
_NSC = 16
_NW = _NC * _NSC
_RPW = (_B * _C) // _NW
_L = 16


def _vq_body(z_ref, cb_ref, idx_ref):
    cb = cb_ref[...]
    esq = jnp.sum(cb * cb, axis=1, keepdims=True)
    kio = jax.lax.broadcasted_iota(jnp.int32, (_K, _HW), 0)
    for i in range(_MB):
        zb = z_ref[i]
        dot = jax.lax.dot_general(
            cb, zb, (((1,), (0,)), ((), ())),
            preferred_element_type=jnp.float32,
        )
        s = zb * zb
        w = _C
        while w > 1:
            w //= 2
            s = s[:w] + s[w:2 * w]
        zsq = s
        d = zsq - 2.0 * dot + esq
        dmin = jnp.min(d, axis=0, keepdims=True)
        idx = jnp.min(jnp.where(d == dmin, kio, _K), axis=0).astype(jnp.int32)
        idx_ref[i] = idx.reshape(8, 128)


def _lookup_body(cb_hbm, idx_hbm, out_hbm, cb_v, idx_v, out_v):
    w = lax.axis_index("s") * _NC + lax.axis_index("c")
    row0 = w * _RPW
    b = row0 // _C
    c0 = row0 % _C
    pltpu.sync_copy(cb_hbm, cb_v)
    pltpu.sync_copy(idx_hbm.at[b], idx_v)
    def cloop(ci, carry):
        def gloop(g, carry2):
            ids = idx_v[pl.ds(g * _L, _L)]
            flat = ids * _C + (c0 + ci)
            vals = plsc.load_gather(cb_v, [flat])
            out_v[ci, pl.ds(g * _L, _L)] = vals
            return carry2
        return lax.fori_loop(0, _HW // _L, gloop, carry)
    lax.fori_loop(0, _RPW, cloop, 0)
    pltpu.sync_copy(out_v, out_hbm.at[pl.ds(row0, _RPW)])


def kernel(z_e, codebook):
    B, C, H, W = z_e.shape
    z = z_e.reshape(B, C, _HW)
    idx = pl.pallas_call(
        _vq_body,
        grid=(B // _MB,),
        in_specs=[
            pl.BlockSpec((_MB, C, _HW), lambda b: (b, 0, 0)),
            pl.BlockSpec((_K, C), lambda b: (0, 0)),
        ],
        out_specs=pl.BlockSpec((_MB, 8, 128), lambda b: (b, 0, 0)),
        out_shape=jax.ShapeDtypeStruct((B, 8, 128), jnp.int32),
        compiler_params=pltpu.CompilerParams(
            dimension_semantics=("arbitrary",),
        ),
    )(z, codebook)

    mesh = plsc.VectorSubcoreMesh(core_axis_name="c", subcore_axis_name="s")
    lookup = functools.partial(
        pl.kernel,
        mesh=mesh,
        out_type=jax.ShapeDtypeStruct((B * C, _HW), jnp.float32),
        scratch_types=[
            pltpu.VMEM((_K * C,), jnp.float32),
            pltpu.VMEM((_HW,), jnp.int32),
            pltpu.VMEM((_RPW, _HW), jnp.float32),
        ],
        compiler_params=pltpu.CompilerParams(
            use_tc_tiling_on_sc=False, needs_layout_passes=False
        ),
    )(_lookup_body)
    zq = lookup(codebook.reshape(_K * C), idx.reshape(B, _HW))
    return zq.reshape(B, C, H, W), idx.reshape(-1)


# speedup vs baseline: 1.1581x; 1.1581x over previous
import functools

import jax
import jax.numpy as jnp
from jax import lax
from jax.experimental import pallas as pl
from jax.experimental.pallas import tpu as pltpu
from jax.experimental.pallas import tpu_sc as plsc

_B, _C, _H, _W = 16, 64, 32, 32
_HW = _H * _W
_K = 512
_MB = 4

_NC = 2
_NSC = 16
_NW = _NC * _NSC
_RPW = (_B * _C) // _NW
_L = 16


def _vq_body(z_ref, cb_ref, idx_ref):
    cb = cb_ref[...]
    esq = jnp.sum(cb * cb, axis=1, keepdims=True)
    kio = jax.lax.broadcasted_iota(jnp.int32, (_K, _HW), 0)
    for i in range(_MB):
        zb = z_ref[i]
        dot = jax.lax.dot_general(
            cb, zb, (((1,), (0,)), ((), ())),
            preferred_element_type=jnp.float32,
        )
        s = zb * zb
        w = _C
        while w > 1:
            w //= 2
            s = s[:w] + s[w:2 * w]
        zsq = s
        d = zsq - 2.0 * dot + esq
        dmin = jnp.min(d, axis=0, keepdims=True)
        idx = jnp.min(jnp.where(d == dmin, kio, _K), axis=0).astype(jnp.int32)
        idx_ref[i] = idx.reshape(8, 128)


def _lookup_body(cb_hbm, idx_hbm, out_hbm, cb_v, idx_v, out_v):
    w = lax.axis_index("s") * _NC + lax.axis_index("c")
    row0 = w * _RPW
    b = row0 // _C
    c0 = row0 % _C
    pltpu.sync_copy(cb_hbm, cb_v)
    pltpu.sync_copy(idx_hbm.at[b], idx_v)
    def gloop(g, carry):
        ids = idx_v[pl.ds(g * _L, _L)]
        base = ids * _C + c0
        for ci in range(_RPW):
            vals = plsc.load_gather(cb_v, [base + ci])
            out_v[ci, pl.ds(g * _L, _L)] = vals
        return carry
    lax.fori_loop(0, _HW // _L, gloop, 0)
    pltpu.sync_copy(out_v, out_hbm.at[pl.ds(row0, _RPW)])


def kernel(z_e, codebook):
    B, C, H, W = z_e.shape
    z = z_e.reshape(B, C, _HW)
    idx = pl.pallas_call(
        _vq_body,
        grid=(B // _MB,),
        in_specs=[
            pl.BlockSpec((_MB, C, _HW), lambda b: (b, 0, 0)),
            pl.BlockSpec((_K, C), lambda b: (0, 0)),
        ],
        out_specs=pl.BlockSpec((_MB, 8, 128), lambda b: (b, 0, 0)),
        out_shape=jax.ShapeDtypeStruct((B, 8, 128), jnp.int32),
        compiler_params=pltpu.CompilerParams(
            dimension_semantics=("arbitrary",),
        ),
    )(z, codebook)

    mesh = plsc.VectorSubcoreMesh(core_axis_name="c", subcore_axis_name="s")
    lookup = functools.partial(
        pl.kernel,
        mesh=mesh,
        out_type=jax.ShapeDtypeStruct((B * C, _HW), jnp.float32),
        scratch_types=[
            pltpu.VMEM((_K * C,), jnp.float32),
            pltpu.VMEM((_HW,), jnp.int32),
            pltpu.VMEM((_RPW, _HW), jnp.float32),
        ],
        compiler_params=pltpu.CompilerParams(
            use_tc_tiling_on_sc=False, needs_layout_passes=False
        ),
    )(_lookup_body)
    zq = lookup(codebook.reshape(_K * C), idx.reshape(B, _HW))
    return zq.reshape(B, C, H, W), idx.reshape(-1)


# speedup vs baseline: 1.3691x; 1.1822x over previous
import functools

import jax
import jax.numpy as jnp
from jax import lax
from jax.experimental import pallas as pl
from jax.experimental.pallas import tpu as pltpu
from jax.experimental.pallas import tpu_sc as plsc

_B, _C, _H, _W = 16, 64, 32, 32
_HW = _H * _W
_K = 512
_MB = 4

_NC = 2
_NSC = 16
_NW = _NC * _NSC
_RPW = (_B * _C) // _NW
_L = 16


def _vq_body(z_ref, cb_ref, idx_ref):
    cb = cb_ref[...]
    esq = jnp.sum(cb * cb, axis=1, keepdims=True)
    kio = jax.lax.broadcasted_iota(jnp.int32, (_K, _HW), 0)
    for i in range(_MB):
        zb = z_ref[i]
        dot = jax.lax.dot_general(
            cb, zb, (((1,), (0,)), ((), ())),
            preferred_element_type=jnp.float32,
        )
        s = zb * zb
        w = _C
        while w > 1:
            w //= 2
            s = s[:w] + s[w:2 * w]
        zsq = s
        d = zsq - 2.0 * dot + esq
        dmin = jnp.min(d, axis=0, keepdims=True)
        idx = jnp.min(jnp.where(d == dmin, kio, _K), axis=0).astype(jnp.int32)
        idx_ref[i] = idx.reshape(8, 128)


def _lookup_body(cb_hbm, idx_hbm, out_hbm, cb_v, idx_v, out_v):
    w = lax.axis_index("s") * _NC + lax.axis_index("c")
    row0 = w * _RPW
    b = row0 // _C
    c0 = row0 % _C
    pltpu.sync_copy(cb_hbm, cb_v)
    pltpu.sync_copy(idx_hbm.at[b], idx_v)
    def gloop(g, carry):
        ids = idx_v[pl.ds(g * _L, _L)]
        base = ids + c0 * _K
        for ci in range(_RPW):
            vals = plsc.load_gather(cb_v, [base + ci * _K])
            out_v[ci, pl.ds(g * _L, _L)] = vals
        return carry
    lax.fori_loop(0, _HW // _L, gloop, 0)
    pltpu.sync_copy(out_v, out_hbm.at[pl.ds(row0, _RPW)])


def kernel(z_e, codebook):
    B, C, H, W = z_e.shape
    z = z_e.reshape(B, C, _HW)
    idx = pl.pallas_call(
        _vq_body,
        grid=(B // _MB,),
        in_specs=[
            pl.BlockSpec((_MB, C, _HW), lambda b: (b, 0, 0)),
            pl.BlockSpec((_K, C), lambda b: (0, 0)),
        ],
        out_specs=pl.BlockSpec((_MB, 8, 128), lambda b: (b, 0, 0)),
        out_shape=jax.ShapeDtypeStruct((B, 8, 128), jnp.int32),
        compiler_params=pltpu.CompilerParams(
            dimension_semantics=("arbitrary",),
        ),
    )(z, codebook)

    mesh = plsc.VectorSubcoreMesh(core_axis_name="c", subcore_axis_name="s")
    lookup = functools.partial(
        pl.kernel,
        mesh=mesh,
        out_type=jax.ShapeDtypeStruct((B * C, _HW), jnp.float32),
        scratch_types=[
            pltpu.VMEM((_K * C,), jnp.float32),
            pltpu.VMEM((_HW,), jnp.int32),
            pltpu.VMEM((_RPW, _HW), jnp.float32),
        ],
        compiler_params=pltpu.CompilerParams(
            use_tc_tiling_on_sc=False, needs_layout_passes=False
        ),
    )(_lookup_body)
    zq = lookup(codebook.T.reshape(_K * C), idx.reshape(B, _HW))
    return zq.reshape(B, C, H, W), idx.reshape(-1)


# speedup vs baseline: 1.4754x; 1.0777x over previous
import functools

import jax
import jax.numpy as jnp
from jax import lax
from jax.experimental import pallas as pl
from jax.experimental.pallas import tpu as pltpu
from jax.experimental.pallas import tpu_sc as plsc

_B, _C, _H, _W = 16, 64, 32, 32
_HW = _H * _W
_K = 512
_MB = 4

_NC = 2
_NSC = 16
_NW = _NC * _NSC
_RPW = (_B * _C) // _NW
_L = 16


def _vq_body(z_ref, cb_ref, idx_ref):
    cb = cb_ref[...]
    esq = jnp.sum(cb * cb, axis=1, keepdims=True)
    kio = jax.lax.broadcasted_iota(jnp.int32, (_K, _HW), 0)
    for i in range(_MB):
        zb = z_ref[i]
        dot = jax.lax.dot_general(
            cb, zb, (((1,), (0,)), ((), ())),
            preferred_element_type=jnp.float32,
        )
        s = zb * zb
        w = _C
        while w > 1:
            w //= 2
            s = s[:w] + s[w:2 * w]
        zsq = s
        d = zsq - 2.0 * dot + esq
        dmin = jnp.min(d, axis=0, keepdims=True)
        idx = jnp.min(jnp.where(d == dmin, kio, _K), axis=0).astype(jnp.int32)
        idx_ref[i] = idx.reshape(8, 128)


def _lookup_body(cb_hbm, idx_hbm, out_hbm, cb_v, idx_v, out_v):
    w = lax.axis_index("s") * _NC + lax.axis_index("c")
    row0 = w * _RPW
    b = row0 // _C
    c0 = row0 % _C
    pltpu.sync_copy(cb_hbm, cb_v)
    pltpu.sync_copy(idx_hbm.at[b], idx_v)
    @plsc.parallel_loop(0, _HW // _L, unroll=2)
    def gloop(g):
        ids = idx_v[pl.ds(g * _L, _L)]
        base = ids + c0 * _K
        for ci in range(_RPW):
            vals = plsc.load_gather(cb_v, [base + ci * _K])
            out_v[ci, pl.ds(g * _L, _L)] = vals
    pltpu.sync_copy(out_v, out_hbm.at[pl.ds(row0, _RPW)])


def kernel(z_e, codebook):
    B, C, H, W = z_e.shape
    z = z_e.reshape(B, C, _HW)
    idx = pl.pallas_call(
        _vq_body,
        grid=(B // _MB,),
        in_specs=[
            pl.BlockSpec((_MB, C, _HW), lambda b: (b, 0, 0)),
            pl.BlockSpec((_K, C), lambda b: (0, 0)),
        ],
        out_specs=pl.BlockSpec((_MB, 8, 128), lambda b: (b, 0, 0)),
        out_shape=jax.ShapeDtypeStruct((B, 8, 128), jnp.int32),
        compiler_params=pltpu.CompilerParams(
            dimension_semantics=("arbitrary",),
        ),
    )(z, codebook)

    mesh = plsc.VectorSubcoreMesh(core_axis_name="c", subcore_axis_name="s")
    lookup = functools.partial(
        pl.kernel,
        mesh=mesh,
        out_type=jax.ShapeDtypeStruct((B * C, _HW), jnp.float32),
        scratch_types=[
            pltpu.VMEM((_K * C,), jnp.float32),
            pltpu.VMEM((_HW,), jnp.int32),
            pltpu.VMEM((_RPW, _HW), jnp.float32),
        ],
        compiler_params=pltpu.CompilerParams(
            use_tc_tiling_on_sc=False, needs_layout_passes=False
        ),
    )(_lookup_body)
    zq = lookup(codebook.T.reshape(_K * C), idx.reshape(B, _HW))
    return zq.reshape(B, C, H, W), idx.reshape(-1)
